# baseline (speedup 1.0000x reference)
import dataclasses
import functools

import jax
import jax.numpy as jnp
from jax import lax
from jax.experimental import pallas as pl
from jax.experimental.pallas import tpu as pltpu
from jax.experimental.pallas import tpu_sc as plsc

_W = 0.01
_NC, _NS = 2, 16
_NW = _NC * _NS
_RB = 16


def _make_sc_stats(rows, cols):
    nr_w = rows // _NW
    nblk = nr_w // _RB
    mesh = plsc.VectorSubcoreMesh(
        core_axis_name="c", subcore_axis_name="s",
        num_cores=_NC, num_subcores=_NS,
    )

    cp = pltpu.CompilerParams()
    if "needs_layout_passes" in pltpu.CompilerParams.__dataclass_fields__:
        cp = dataclasses.replace(cp, needs_layout_passes=False)

    @functools.partial(
        pl.kernel,
        compiler_params=cp,
        out_type=[
            jax.ShapeDtypeStruct((rows,), jnp.float32),
            jax.ShapeDtypeStruct((rows,), jnp.float32),
            jax.ShapeDtypeStruct((rows,), jnp.float32),
        ],
        mesh=mesh,
        scratch_types=[
            pltpu.VMEM((2, _RB, cols), jnp.float32),
            pltpu.VMEM((2, _RB, cols), jnp.int32),
            pltpu.VMEM((nr_w,), jnp.float32),
            pltpu.VMEM((nr_w,), jnp.float32),
            pltpu.VMEM((nr_w,), jnp.float32),
            pltpu.SemaphoreType.DMA((2, 2)),
        ],
    )
    def sc_stats(x_hbm, t_hbm, d_hbm, s_hbm, c_hbm,
                 xb, tb, dst, sst, cst, sems):
        w = lax.axis_index("s") * _NC + lax.axis_index("c")
        base = w * nr_w

        def issue(k, slot):
            pltpu.async_copy(
                x_hbm.at[pl.ds(base + k * _RB, _RB)], xb.at[slot],
                sems.at[slot, 0])
            pltpu.async_copy(
                t_hbm.at[pl.ds(base + k * _RB, _RB)], tb.at[slot],
                sems.at[slot, 1])

        issue(0, 0)
        row_i = lax.iota(jnp.int32, 16)
        zero = jnp.zeros((16,), jnp.float32)

        for k in range(nblk):
            slot = k % 2
            pltpu.make_async_copy(
                x_hbm.at[pl.ds(base + k * _RB, _RB)], xb.at[slot],
                sems.at[slot, 0]).wait()
            pltpu.make_async_copy(
                t_hbm.at[pl.ds(base + k * _RB, _RB)], tb.at[slot],
                sems.at[slot, 1]).wait()
            if k + 1 < nblk:
                issue(k + 1, (k + 1) % 2)

            xr = xb.at[slot]
            tr = tb.at[slot]

            def jstep(j, carry, xr=xr, tr=tr):
                d, s, c = carry
                for u in range(4):
                    colv = jnp.full((16,), j * 4 + u, jnp.int32)
                    xv = plsc.load_gather(xr, [row_i, colv])
                    tv = plsc.load_gather(tr, [row_i, colv])
                    mask = tv != 0
                    xm = jnp.where(mask, xv, -10000.0)
                    e = jnp.exp(xm)
                    d = d + e
                    s = s + e * xm
                    c = c + jnp.where(mask, 1.0, 0.0)
                return d, s, c

            d, s, c = lax.fori_loop(0, cols // 4, jstep, (zero, zero, zero))
            dst[pl.ds(k * _RB, _RB)] = d
            sst[pl.ds(k * _RB, _RB)] = s
            cst[pl.ds(k * _RB, _RB)] = c

        pltpu.sync_copy(dst, d_hbm.at[pl.ds(base, nr_w)])
        pltpu.sync_copy(sst, s_hbm.at[pl.ds(base, nr_w)])
        pltpu.sync_copy(cst, c_hbm.at[pl.ds(base, nr_w)])

    return sc_stats


def _combine_body(d_ref, s_ref, c_ref, out_ref):
    d = d_ref[...]
    s = s_ref[...]
    c = c_ref[...]
    dsafe = jnp.where(c > 0.0, d, 1.0)
    contrib = jnp.where(c > 0.0, jnp.log(dsafe) - s / dsafe, 0.0)
    out_ref[0, 0] = _W * jnp.sum(contrib) / jnp.sum(c)


def kernel(logits, target):
    rows, cols = logits.shape
    d, s, c = _make_sc_stats(rows, cols)(logits, target)
    side = 128
    d2 = d.reshape(rows // side, side)
    s2 = s.reshape(rows // side, side)
    c2 = c.reshape(rows // side, side)
    out = pl.pallas_call(
        _combine_body,
        out_specs=pl.BlockSpec(memory_space=pltpu.SMEM),
        out_shape=jax.ShapeDtypeStruct((1, 1), jnp.float32),
    )(d2, s2, c2)
    return out[0, 0]


# speedup vs baseline: 2.6770x; 2.6770x over previous
import dataclasses
import functools

import jax
import jax.numpy as jnp
from jax import lax
from jax.experimental import pallas as pl
from jax.experimental.pallas import tpu as pltpu
from jax.experimental.pallas import tpu_sc as plsc

_W = 0.01
_NC, _NS = 2, 16
_NW = _NC * _NS
_RB = 16


def _make_sc_stats(rows, cols):
    nr_w = rows // _NW
    nblk = nr_w // _RB
    mesh = plsc.VectorSubcoreMesh(
        core_axis_name="c", subcore_axis_name="s",
        num_cores=_NC, num_subcores=_NS,
    )

    cp = pltpu.CompilerParams()
    if "needs_layout_passes" in pltpu.CompilerParams.__dataclass_fields__:
        cp = dataclasses.replace(cp, needs_layout_passes=False)

    @functools.partial(
        pl.kernel,
        compiler_params=cp,
        out_type=[
            jax.ShapeDtypeStruct((rows,), jnp.float32),
            jax.ShapeDtypeStruct((rows,), jnp.float32),
            jax.ShapeDtypeStruct((rows,), jnp.float32),
        ],
        mesh=mesh,
        scratch_types=[
            pltpu.VMEM((2, _RB, cols), jnp.float32),
            pltpu.VMEM((2, _RB, cols), jnp.int32),
            pltpu.VMEM((nr_w,), jnp.float32),
            pltpu.VMEM((nr_w,), jnp.float32),
            pltpu.VMEM((nr_w,), jnp.float32),
            pltpu.SemaphoreType.DMA((2, 2)),
        ],
    )
    def sc_stats(x_hbm, t_hbm, d_hbm, s_hbm, c_hbm,
                 xb, tb, dst, sst, cst, sems):
        w = lax.axis_index("s") * _NC + lax.axis_index("c")
        base = w * nr_w

        def issue(k, slot):
            pltpu.async_copy(
                x_hbm.at[pl.ds(base + k * _RB, _RB)], xb.at[slot],
                sems.at[slot, 0])
            pltpu.async_copy(
                t_hbm.at[pl.ds(base + k * _RB, _RB)], tb.at[slot],
                sems.at[slot, 1])

        issue(0, 0)
        issue(1, 1)
        row_i = lax.iota(jnp.int32, 16)
        zero = jnp.zeros((16,), jnp.float32)
        izero = jnp.zeros((16,), jnp.int32)

        def block(k, slot):
            pltpu.make_async_copy(
                x_hbm.at[pl.ds(base + k * _RB, _RB)], xb.at[slot],
                sems.at[slot, 0]).wait()
            pltpu.make_async_copy(
                t_hbm.at[pl.ds(base + k * _RB, _RB)], tb.at[slot],
                sems.at[slot, 1]).wait()

            xr = xb.at[slot]
            tr = tb.at[slot]

            def jstep(j, carry):
                a = list(carry)
                for u in range(8):
                    g = u % 4
                    b = j * 8 + u
                    cv0 = jnp.full((16,), b, jnp.int32) + row_i
                    cv1 = jnp.full((16,), b - cols, jnp.int32) + row_i
                    colv = jnp.where(cv0 >= cols, cv1, cv0)
                    xv = plsc.load_gather(xr, [row_i, colv])
                    tv = plsc.load_gather(tr, [row_i, colv])
                    mask = tv != 0
                    xm = jnp.where(mask, xv, -10000.0)
                    e = jnp.exp(xm)
                    a[g] = a[g] + e
                    a[4 + g] = a[4 + g] + e * xm
                    a[8 + g] = a[8 + g] + tv
                return tuple(a)

            init = (zero,) * 4 + (zero,) * 4 + (izero,) * 4
            a = lax.fori_loop(0, cols // 8, jstep, init)
            d = (a[0] + a[1]) + (a[2] + a[3])
            s = (a[4] + a[5]) + (a[6] + a[7])
            ci = (a[8] + a[9]) + (a[10] + a[11])
            dst[pl.ds(k * _RB, _RB)] = d
            sst[pl.ds(k * _RB, _RB)] = s
            cst[pl.ds(k * _RB, _RB)] = ci.astype(jnp.float32)

            @pl.when(k + 2 < nblk)
            def _():
                issue(k + 2, slot)

        def pairstep(p, _):
            for slot in (0, 1):
                block(p * 2 + slot, slot)
            return 0

        lax.fori_loop(0, nblk // 2, pairstep, 0)

        pltpu.sync_copy(dst, d_hbm.at[pl.ds(base, nr_w)])
        pltpu.sync_copy(sst, s_hbm.at[pl.ds(base, nr_w)])
        pltpu.sync_copy(cst, c_hbm.at[pl.ds(base, nr_w)])

    return sc_stats


def _combine_body(d_ref, s_ref, c_ref, out_ref):
    d = d_ref[...]
    s = s_ref[...]
    c = c_ref[...]
    dsafe = jnp.where(c > 0.0, d, 1.0)
    contrib = jnp.where(c > 0.0, jnp.log(dsafe) - s / dsafe, 0.0)
    out_ref[0, 0] = _W * jnp.sum(contrib) / jnp.sum(c)


def kernel(logits, target):
    rows, cols = logits.shape
    d, s, c = _make_sc_stats(rows, cols)(logits, target)
    side = 128
    d2 = d.reshape(rows // side, side)
    s2 = s.reshape(rows // side, side)
    c2 = c.reshape(rows // side, side)
    out = pl.pallas_call(
        _combine_body,
        out_specs=pl.BlockSpec(memory_space=pltpu.SMEM),
        out_shape=jax.ShapeDtypeStruct((1, 1), jnp.float32),
    )(d2, s2, c2)
    return out[0, 0]
